# Initial kernel scaffold; baseline (speedup 1.0000x reference)
import jax
import jax.numpy as jnp
from jax.experimental import pallas as pl


def kernel(x, edge_index, edge_type, node_emb, rel_emb1, W_rel1, b_rel1, W_node1, b_node1, g1, be1, rel_emb2, W_rel2, b_rel2, W_node2, b_node2, g2, be2):
    raise NotImplementedError("write your pallas kernel here")


# speedup vs baseline: 3.6289x; 3.6289x over previous
import jax
import jax.numpy as jnp
from jax import lax
from jax.experimental import pallas as pl
from jax.experimental.pallas import tpu as pltpu
from jax.experimental.pallas import tpu_sc as plsc

N = 10000
E = 320000
D = 128
TWO_R = 1000

NC = 2
NS = 16
NW = NC * NS
EPW = E // NW
CH = 80
NCHUNK = EPW // CH
NPAD = 10240
RPT = NPAD // NS
ZR = 160
LN_EPS = 1e-5


def _sc_layer_body(h_hbm, rel_hbm, src_hbm, dst_hbm, ety_hbm, out_hbm,
                   srcb, dstb, etyb, hb, rb, zb, agg_sh, sem1, sem2):
    c = lax.axis_index("c")
    s = lax.axis_index("s")
    wid = s * NC + c

    def zrow(r, carry):
        for j in range(D // 16):
            zb[r, pl.ds(j * 16, 16)] = jnp.zeros((16,), jnp.float32)
        return carry
    lax.fori_loop(0, ZR, zrow, 0)
    row0 = s * RPT
    for k in range(RPT // ZR):
        pltpu.sync_copy(zb, agg_sh.at[pl.ds(row0 + k * ZR, ZR)])
    plsc.subcore_barrier()

    base0 = wid * EPW

    def chunk(k, carry):
        base = base0 + k * CH
        pltpu.sync_copy(src_hbm.at[pl.ds(base, CH)], srcb)
        pltpu.sync_copy(ety_hbm.at[pl.ds(base, CH)], etyb)
        pltpu.sync_copy(dst_hbm.at[pl.ds(base, CH)], dstb)
        cp1 = pltpu.async_copy(h_hbm.at[srcb], hb, sem1)
        cp2 = pltpu.async_copy(rel_hbm.at[etyb], rb, sem2)
        cp1.wait()
        cp2.wait()

        def row(r, rc):
            for j in range(D // 16):
                sl = pl.ds(j * 16, 16)
                hb[r, sl] = hb[r, sl] * rb[r, sl]
            return rc
        lax.fori_loop(0, CH, row, 0)
        pltpu.sync_copy(hb, agg_sh.at[dstb], add=True)
        return carry

    lax.fori_loop(0, NCHUNK, chunk, 0)
    plsc.subcore_barrier()

    pltpu.sync_copy(agg_sh.at[pl.ds(row0, RPT)],
                    out_hbm.at[pl.ds(c * NPAD + row0, RPT)])


_sc_layer = pl.kernel(
    _sc_layer_body,
    out_type=jax.ShapeDtypeStruct((2 * NPAD, D), jnp.float32),
    mesh=plsc.VectorSubcoreMesh(core_axis_name="c", subcore_axis_name="s",
                                num_cores=NC, num_subcores=NS),
    scratch_types=[
        pltpu.VMEM((CH,), jnp.int32),
        pltpu.VMEM((CH,), jnp.int32),
        pltpu.VMEM((CH,), jnp.int32),
        pltpu.VMEM((CH, D), jnp.float32),
        pltpu.VMEM((CH, D), jnp.float32),
        pltpu.VMEM((ZR, D), jnp.float32),
        pltpu.VMEM_SHARED((NPAD, D), jnp.float32),
        pltpu.SemaphoreType.DMA,
        pltpu.SemaphoreType.DMA,
    ],
)


def _dense_body(pa, pb, wt, b, g, be, o):
    x = pa[...] + pb[...]
    y = jnp.dot(x, wt[...], preferred_element_type=jnp.float32) + b[...]
    mu = jnp.mean(y, axis=1, keepdims=True)
    var = jnp.mean(jnp.square(y - mu), axis=1, keepdims=True)
    y = (y - mu) * lax.rsqrt(var + LN_EPS) * g[...] + be[...]
    o[...] = jnp.maximum(y, 0.0)


_BR = 1000


def _dense(parts, wt, b, g, be):
    return pl.pallas_call(
        _dense_body,
        out_shape=jax.ShapeDtypeStruct((N, D), jnp.float32),
        grid=(N // _BR,),
        in_specs=[
            pl.BlockSpec((_BR, D), lambda i: (i, 0)),
            pl.BlockSpec((_BR, D), lambda i: (i, 0)),
            pl.BlockSpec((D, D), lambda i: (0, 0)),
            pl.BlockSpec((1, D), lambda i: (0, 0)),
            pl.BlockSpec((1, D), lambda i: (0, 0)),
            pl.BlockSpec((1, D), lambda i: (0, 0)),
        ],
        out_specs=pl.BlockSpec((_BR, D), lambda i: (i, 0)),
    )(parts[:N], parts[NPAD:NPAD + N], wt, b, g, be)


def _rels_body(r, wt, b, o):
    o[...] = jnp.dot(r[...], wt[...], preferred_element_type=jnp.float32) + b[...]


def _rels(rel_emb, wt, b):
    return pl.pallas_call(
        _rels_body,
        out_shape=jax.ShapeDtypeStruct((TWO_R, D), jnp.float32),
    )(rel_emb, wt, b)


def kernel(x, edge_index, edge_type, node_emb, rel_emb1, W_rel1, b_rel1,
           W_node1, b_node1, g1, be1, rel_emb2, W_rel2, b_rel2,
           W_node2, b_node2, g2, be2):
    src = edge_index[0].astype(jnp.int32)
    dst = edge_index[1].astype(jnp.int32)
    ety = edge_type.astype(jnp.int32)
    h0 = jnp.take(node_emb, x.astype(jnp.int32), axis=0)

    p1 = _sc_layer(h0, rel_emb1, src, dst, ety)
    h1 = _dense(p1, W_node1.T, b_node1[None, :], g1[None, :], be1[None, :])
    p2 = _sc_layer(h1, rel_emb2, src, dst, ety)
    h2 = _dense(p2, W_node2.T, b_node2[None, :], g2[None, :], be2[None, :])
    rels = _rels(rel_emb2, W_rel2.T, b_rel2[None, :])
    return (h2, rels)


# speedup vs baseline: 7.5322x; 2.0756x over previous
import jax
import jax.numpy as jnp
from jax import lax
from jax.experimental import pallas as pl
from jax.experimental.pallas import tpu as pltpu
from jax.experimental.pallas import tpu_sc as plsc

N = 10000
E = 320000
D = 128
TWO_R = 1000

NC = 2
NS = 16
NW = NC * NS
EPW = E // NW
CH = 40
NCHUNK = EPW // CH
NPAD = 10112
RPT = NPAD // NS
ZR = 8
LAST = NCHUNK - 1
LN_EPS = 1e-5

ND = 8


def _sc_layer_body(h_hbm, rel_hbm, src_hbm, dst_hbm, ety_hbm, out_hbm, *scr):
    sidx = scr[0:ND]
    eidx = scr[ND:2 * ND]
    didx = scr[2 * ND:3 * ND]
    hbufs = scr[3 * ND:3 * ND + 2]
    rbufs = scr[3 * ND + 2:3 * ND + 4]
    obufs = scr[3 * ND + 4:3 * ND + 6]
    zb = scr[3 * ND + 6]
    agg_sh = scr[3 * ND + 7]
    isem = scr[3 * ND + 8:3 * ND + 8 + ND]
    gsem = scr[3 * ND + 8 + ND:3 * ND + 10 + ND]
    ssem = scr[3 * ND + 10 + ND:3 * ND + 12 + ND]

    c = lax.axis_index("c")
    s = lax.axis_index("s")
    wid = s * NC + c

    for r in range(ZR):
        for j in range(D // 16):
            zb[r, pl.ds(j * 16, 16)] = jnp.zeros((16,), jnp.float32)
    row0 = s * RPT

    def zcp(k, carry):
        pltpu.sync_copy(zb, agg_sh.at[pl.ds(
            pl.multiple_of(row0 + k * ZR, 8), ZR)])
        return carry
    lax.fori_loop(0, RPT // ZR, zcp, 0)
    plsc.subcore_barrier()

    base0 = wid * EPW

    def pre_idx_sync(k, q):
        pltpu.sync_copy(src_hbm.at[pl.ds(base0 + k * CH, CH)], sidx[q])
        pltpu.sync_copy(ety_hbm.at[pl.ds(base0 + k * CH, CH)], eidx[q])
        pltpu.sync_copy(dst_hbm.at[pl.ds(base0 + k * CH, CH)], didx[q])

    def pre_idx(k, q):
        pltpu.async_copy(src_hbm.at[pl.ds(base0 + k * CH, CH)],
                         sidx[q], isem[q])
        pltpu.async_copy(ety_hbm.at[pl.ds(base0 + k * CH, CH)],
                         eidx[q], isem[q])
        pltpu.async_copy(dst_hbm.at[pl.ds(base0 + k * CH, CH)],
                         didx[q], isem[q])

    def drain_idx(k, q):
        pltpu.make_async_copy(src_hbm.at[pl.ds(base0 + k * CH, CH)],
                              sidx[q], isem[q]).wait()
        pltpu.make_async_copy(ety_hbm.at[pl.ds(base0 + k * CH, CH)],
                              eidx[q], isem[q]).wait()
        pltpu.make_async_copy(dst_hbm.at[pl.ds(base0 + k * CH, CH)],
                              didx[q], isem[q]).wait()

    def issue_gather(q, b):
        pltpu.async_copy(h_hbm.at[sidx[q]], hbufs[b], gsem[b])
        pltpu.async_copy(rel_hbm.at[eidx[q]], rbufs[b], gsem[b])

    def drain_gather(q, b):
        pltpu.make_async_copy(h_hbm.at[sidx[q]], hbufs[b], gsem[b]).wait()
        pltpu.make_async_copy(rel_hbm.at[eidx[q]], rbufs[b], gsem[b]).wait()

    def wait_scatter(q, b):
        pltpu.make_async_copy(obufs[b], agg_sh.at[didx[q]], ssem[b]).wait()

    def multiply(q, b):
        hb, rb, ob = hbufs[b], rbufs[b], obufs[b]

        def row(r, rc):
            for j in range(D // 16):
                sl = pl.ds(j * 16, 16)
                ob[r, sl] = hb[r, sl] * rb[r, sl]
            return rc
        lax.fori_loop(0, CH, row, 0)

    def start_scatter(q, b):
        pltpu.async_copy(obufs[b], agg_sh.at[didx[q]], ssem[b], add=True)

    for q in range(4):
        pre_idx_sync(q, q)
    for b in range(2):
        issue_gather(b, b)

    def outer(i, carry):
        for u in range(ND):
            k = ND * i + u
            b = u % 2
            drain_gather(u, b)
            multiply(u, b)
            @pl.when(k >= 2)
            def _():
                wait_scatter((u + ND - 2) % ND, b)
            start_scatter(u, b)
            @pl.when(k >= 2)
            def _():
                drain_idx(k + 2, (u + 2) % ND)
            issue_gather((u + 2) % ND, b)
            @pl.when(k + 4 <= LAST)
            def _():
                pre_idx(k + 4, (u + 4) % ND)
        return carry
    lax.fori_loop(0, NCHUNK // ND, outer, 0)

    for t in range(2):
        k = NCHUNK - 2 + t
        u = k % ND
        b = u % 2
        drain_gather(u, b)
        multiply(u, b)
        wait_scatter((u + ND - 2) % ND, b)
        start_scatter(u, b)
    for t in range(2):
        k = NCHUNK - 2 + t
        u = k % ND
        wait_scatter(u, u % 2)

    plsc.subcore_barrier()
    pltpu.sync_copy(agg_sh.at[pl.ds(row0, RPT)],
                    out_hbm.at[pl.ds(c * NPAD + row0, RPT)])


_sc_layer = pl.kernel(
    _sc_layer_body,
    out_type=jax.ShapeDtypeStruct((2 * NPAD, D), jnp.float32),
    mesh=plsc.VectorSubcoreMesh(core_axis_name="c", subcore_axis_name="s",
                                num_cores=NC, num_subcores=NS),
    scratch_types=(
        [pltpu.VMEM((CH,), jnp.int32) for _ in range(3 * ND)]
        + [pltpu.VMEM((CH, D), jnp.float32) for _ in range(6)]
        + [pltpu.VMEM((ZR, D), jnp.float32),
           pltpu.VMEM_SHARED((NPAD, D), jnp.float32)]
        + [pltpu.SemaphoreType.DMA for _ in range(ND + 4)]
    ),
)


def _dense_body(pa, pb, wt, b, g, be, o):
    x = pa[...] + pb[...]
    y = jnp.dot(x, wt[...], preferred_element_type=jnp.float32) + b[...]
    mu = jnp.mean(y, axis=1, keepdims=True)
    var = jnp.mean(jnp.square(y - mu), axis=1, keepdims=True)
    y = (y - mu) * lax.rsqrt(var + LN_EPS) * g[...] + be[...]
    o[...] = jnp.maximum(y, 0.0)


_BR = 1000


def _dense(parts, wt, b, g, be):
    return pl.pallas_call(
        _dense_body,
        out_shape=jax.ShapeDtypeStruct((N, D), jnp.float32),
        grid=(N // _BR,),
        in_specs=[
            pl.BlockSpec((_BR, D), lambda i: (i, 0)),
            pl.BlockSpec((_BR, D), lambda i: (i, 0)),
            pl.BlockSpec((D, D), lambda i: (0, 0)),
            pl.BlockSpec((1, D), lambda i: (0, 0)),
            pl.BlockSpec((1, D), lambda i: (0, 0)),
            pl.BlockSpec((1, D), lambda i: (0, 0)),
        ],
        out_specs=pl.BlockSpec((_BR, D), lambda i: (i, 0)),
    )(parts[:N], parts[NPAD:NPAD + N], wt, b, g, be)


def _rels_body(r, wt, b, o):
    o[...] = jnp.dot(r[...], wt[...], preferred_element_type=jnp.float32) + b[...]


def _rels(rel_emb, wt, b):
    return pl.pallas_call(
        _rels_body,
        out_shape=jax.ShapeDtypeStruct((TWO_R, D), jnp.float32),
    )(rel_emb, wt, b)


def kernel(x, edge_index, edge_type, node_emb, rel_emb1, W_rel1, b_rel1,
           W_node1, b_node1, g1, be1, rel_emb2, W_rel2, b_rel2,
           W_node2, b_node2, g2, be2):
    src = edge_index[0].astype(jnp.int32)
    dst = edge_index[1].astype(jnp.int32)
    ety = edge_type.astype(jnp.int32)
    h0 = jnp.take(node_emb, x.astype(jnp.int32), axis=0)

    p1 = _sc_layer(h0, rel_emb1, src, dst, ety)
    h1 = _dense(p1, W_node1.T, b_node1[None, :], g1[None, :], be1[None, :])
    p2 = _sc_layer(h1, rel_emb2, src, dst, ety)
    h2 = _dense(p2, W_node2.T, b_node2[None, :], g2[None, :], be2[None, :])
    rels = _rels(rel_emb2, W_rel2.T, b_rel2[None, :])
    return (h2, rels)


# speedup vs baseline: 7.5408x; 1.0011x over previous
import jax
import jax.numpy as jnp
from jax import lax
from jax.experimental import pallas as pl
from jax.experimental.pallas import tpu as pltpu
from jax.experimental.pallas import tpu_sc as plsc

N = 10000
E = 320000
D = 128
TWO_R = 1000

NC = 2
NS = 16
NW = NC * NS
EPW = E // NW
CH = 40
NCHUNK = EPW // CH
NPAD = 10112
RPT = NPAD // NS
ZR = 8
LAST = NCHUNK - 1
LN_EPS = 1e-5

ND = 8


def _sc_layer_body(h_hbm, rel_hbm, src_hbm, dst_hbm, ety_hbm, out_hbm, *scr):
    sidx = scr[0:ND]
    eidx = scr[ND:2 * ND]
    didx = scr[2 * ND:3 * ND]
    hbufs = scr[3 * ND:3 * ND + 2]
    rbufs = scr[3 * ND + 2:3 * ND + 4]
    obufs = scr[3 * ND + 4:3 * ND + 6]
    zb = scr[3 * ND + 6]
    agg_sh = scr[3 * ND + 7]
    isem = scr[3 * ND + 8:3 * ND + 8 + ND]
    gsem = scr[3 * ND + 8 + ND:3 * ND + 10 + ND]
    ssem = scr[3 * ND + 10 + ND:3 * ND + 12 + ND]

    c = lax.axis_index("c")
    s = lax.axis_index("s")
    wid = s * NC + c

    for r in range(ZR):
        for j in range(D // 16):
            zb[r, pl.ds(j * 16, 16)] = jnp.zeros((16,), jnp.float32)
    row0 = s * RPT

    def zcp(k, carry):
        pltpu.sync_copy(zb, agg_sh.at[pl.ds(
            pl.multiple_of(row0 + k * ZR, 8), ZR)])
        return carry
    lax.fori_loop(0, RPT // ZR, zcp, 0)
    plsc.subcore_barrier()

    base0 = wid * EPW

    def pre_idx_sync(k, q):
        pltpu.sync_copy(src_hbm.at[pl.ds(base0 + k * CH, CH)], sidx[q])
        pltpu.sync_copy(ety_hbm.at[pl.ds(base0 + k * CH, CH)], eidx[q])
        pltpu.sync_copy(dst_hbm.at[pl.ds(base0 + k * CH, CH)], didx[q])

    def pre_idx(k, q):
        pltpu.async_copy(src_hbm.at[pl.ds(base0 + k * CH, CH)],
                         sidx[q], isem[q])
        pltpu.async_copy(ety_hbm.at[pl.ds(base0 + k * CH, CH)],
                         eidx[q], isem[q])
        pltpu.async_copy(dst_hbm.at[pl.ds(base0 + k * CH, CH)],
                         didx[q], isem[q])

    def drain_idx(k, q):
        pltpu.make_async_copy(src_hbm.at[pl.ds(base0 + k * CH, CH)],
                              sidx[q], isem[q]).wait()
        pltpu.make_async_copy(ety_hbm.at[pl.ds(base0 + k * CH, CH)],
                              eidx[q], isem[q]).wait()
        pltpu.make_async_copy(dst_hbm.at[pl.ds(base0 + k * CH, CH)],
                              didx[q], isem[q]).wait()

    def issue_gather(q, b):
        pltpu.async_copy(h_hbm.at[sidx[q]], hbufs[b], gsem[b])
        pltpu.async_copy(rel_hbm.at[eidx[q]], rbufs[b], gsem[b])

    def drain_gather(q, b):
        pltpu.make_async_copy(h_hbm.at[sidx[q]], hbufs[b], gsem[b]).wait()
        pltpu.make_async_copy(rel_hbm.at[eidx[q]], rbufs[b], gsem[b]).wait()

    def wait_scatter(q, b):
        pltpu.make_async_copy(obufs[b], agg_sh.at[didx[q]], ssem[b]).wait()

    def multiply(q, b):
        hb, rb, ob = hbufs[b], rbufs[b], obufs[b]

        def row(r, rc):
            for j in range(D // 16):
                sl = pl.ds(j * 16, 16)
                ob[r, sl] = hb[r, sl] * rb[r, sl]
            return rc
        lax.fori_loop(0, CH, row, 0)

    def start_scatter(q, b):
        pltpu.async_copy(obufs[b], agg_sh.at[didx[q]], ssem[b], add=True)

    for q in range(4):
        pre_idx_sync(q, q)
    for b in range(2):
        issue_gather(b, b)

    def outer(i, carry):
        for u in range(ND):
            k = ND * i + u
            b = u % 2
            drain_gather(u, b)
            multiply(u, b)
            @pl.when(k >= 2)
            def _():
                wait_scatter((u + ND - 2) % ND, b)
            start_scatter(u, b)
            @pl.when(k >= 2)
            def _():
                drain_idx(k + 2, (u + 2) % ND)
            issue_gather((u + 2) % ND, b)
            @pl.when(k + 4 <= LAST)
            def _():
                pre_idx(k + 4, (u + 4) % ND)
        return carry
    lax.fori_loop(0, NCHUNK // ND, outer, 0)

    for t in range(2):
        k = NCHUNK - 2 + t
        u = k % ND
        b = u % 2
        drain_gather(u, b)
        multiply(u, b)
        wait_scatter((u + ND - 2) % ND, b)
        start_scatter(u, b)
    for t in range(2):
        k = NCHUNK - 2 + t
        u = k % ND
        wait_scatter(u, u % 2)

    plsc.subcore_barrier()
    pltpu.sync_copy(agg_sh.at[pl.ds(row0, RPT)],
                    out_hbm.at[pl.ds(c * NPAD + row0, RPT)])


_sc_layer = pl.kernel(
    _sc_layer_body,
    out_type=jax.ShapeDtypeStruct((2 * NPAD, D), jnp.float32),
    mesh=plsc.VectorSubcoreMesh(core_axis_name="c", subcore_axis_name="s",
                                num_cores=NC, num_subcores=NS),
    scratch_types=(
        [pltpu.VMEM((CH,), jnp.int32) for _ in range(3 * ND)]
        + [pltpu.VMEM((CH, D), jnp.float32) for _ in range(6)]
        + [pltpu.VMEM((ZR, D), jnp.float32),
           pltpu.VMEM_SHARED((NPAD, D), jnp.float32)]
        + [pltpu.SemaphoreType.DMA for _ in range(ND + 4)]
    ),
)


def _dense_body(pa, pb, wt, b, g, be, o):
    x = pa[...] + pb[...]
    y = jnp.dot(x, wt[...], preferred_element_type=jnp.float32) + b[...]
    mu = jnp.mean(y, axis=1, keepdims=True)
    var = jnp.mean(jnp.square(y - mu), axis=1, keepdims=True)
    y = (y - mu) * lax.rsqrt(var + LN_EPS) * g[...] + be[...]
    o[...] = jnp.maximum(y, 0.0)


_BR = 1000


def _dense(parts, wt, b, g, be):
    return pl.pallas_call(
        _dense_body,
        out_shape=jax.ShapeDtypeStruct((N, D), jnp.float32),
        grid=(N // _BR,),
        in_specs=[
            pl.BlockSpec((_BR, D), lambda i: (i, 0)),
            pl.BlockSpec((_BR, D), lambda i: (i, 0)),
            pl.BlockSpec((D, D), lambda i: (0, 0)),
            pl.BlockSpec((1, D), lambda i: (0, 0)),
            pl.BlockSpec((1, D), lambda i: (0, 0)),
            pl.BlockSpec((1, D), lambda i: (0, 0)),
        ],
        out_specs=pl.BlockSpec((_BR, D), lambda i: (i, 0)),
    )(parts[:N], parts[NPAD:NPAD + N], wt, b, g, be)


def _rels_body(r, wt, b, o):
    o[...] = jnp.dot(r[...], wt[...], preferred_element_type=jnp.float32) + b[...]


def _rels(rel_emb, wt, b):
    return pl.pallas_call(
        _rels_body,
        out_shape=jax.ShapeDtypeStruct((TWO_R, D), jnp.float32),
    )(rel_emb, wt, b)


def kernel(x, edge_index, edge_type, node_emb, rel_emb1, W_rel1, b_rel1,
           W_node1, b_node1, g1, be1, rel_emb2, W_rel2, b_rel2,
           W_node2, b_node2, g2, be2):
    src = edge_index[0].astype(jnp.int32)
    dst = edge_index[1].astype(jnp.int32)
    ety = edge_type.astype(jnp.int32)
    h0 = jnp.take(node_emb, x.astype(jnp.int32), axis=0)

    rels = _rels(rel_emb2, W_rel2.T, b_rel2[None, :])
    p1 = _sc_layer(h0, rel_emb1, src, dst, ety)
    h1 = _dense(p1, W_node1.T, b_node1[None, :], g1[None, :], be1[None, :])
    p2 = _sc_layer(h1, rel_emb2, src, dst, ety)
    h2 = _dense(p2, W_node2.T, b_node2[None, :], g2[None, :], be2[None, :])
    return (h2, rels)
